# Initial kernel scaffold; baseline (speedup 1.0000x reference)
import jax
import jax.numpy as jnp
from jax.experimental import pallas as pl


def kernel(x, psm, mask):
    raise NotImplementedError("write your pallas kernel here")


# speedup vs baseline: 1.1950x; 1.1950x over previous
import functools
import math

import jax
import jax.numpy as jnp
from jax.experimental import pallas as pl
from jax.experimental.pallas import tpu as pltpu

_THRESHOLD = 0.1
_BLK = 128


def _mask_kernel(ego_ref, cavc_ref, mask_ref, f_sc, *, s_total, k):
    def _bf(v):
        return v.astype(jnp.bfloat16).astype(jnp.float32)

    ego = ego_ref[0, 0]
    cavc = cavc_ref[0, 0]
    u_bf = _bf(cavc[:, 0:1])
    v_bf = _bf(cavc[:, 1:2])
    inv_sqrt_c = jnp.float32(math.sqrt(2.0))

    for r in range(s_total // _BLK):
        p0 = r * _BLK
        a = _bf(ego[0:1, p0:p0 + _BLK])
        b = _bf(ego[1:2, p0:p0 + _BLK])
        sim = (u_bf * a + v_bf * b) / inv_sqrt_c
        m = jnp.max(sim, axis=0, keepdims=True)
        e = jnp.exp(sim - m)
        den = jnp.sum(e, axis=0, keepdims=True)
        attn = _bf(e / den)
        y0 = jnp.sum(attn * u_bf, axis=0, keepdims=True)
        y1 = jnp.sum(attn * v_bf, axis=0, keepdims=True)
        z = jnp.maximum(y0, y1)
        f_sc[0:1, p0:p0 + _BLK] = jax.nn.sigmoid(z)

    f = f_sc[0:1, :]
    keys = jax.lax.bitcast_convert_type(f, jnp.int32)
    t = jnp.int32(0)
    for bit in range(30, -1, -1):
        cand = t | jnp.int32(1 << bit)
        cnt = jnp.sum((keys >= cand).astype(jnp.int32))
        t = jnp.where(cnt >= k, cand, t)
    mask_ref[0, 0] = (keys >= t).astype(jnp.float32)


def _compute_masks(psm):
    B, L, C2, H, W = psm.shape
    S = H * W
    psm_r = psm.reshape(B, L, C2, S)
    psm_c = jnp.swapaxes(psm_r, -1, -2)
    k = max(1, int(S * _THRESHOLD))
    kern = functools.partial(_mask_kernel, s_total=S, k=k)
    return pl.pallas_call(
        kern,
        grid=(B, L - 1),
        in_specs=[
            pl.BlockSpec((1, 1, C2, S), lambda b, j: (b, 0, 0, 0)),
            pl.BlockSpec((1, 1, S, C2), lambda b, j: (b, j + 1, 0, 0)),
        ],
        out_specs=pl.BlockSpec((1, 1, 1, S), lambda b, j: (b, j, 0, 0)),
        out_shape=jax.ShapeDtypeStruct((B, L - 1, 1, S), jnp.float32),
        scratch_shapes=[pltpu.VMEM((1, S), jnp.float32)],
    )(psm_r, psm_c)


def _apply_kernel(x_ref, m_ref, o_ref):
    o_ref[...] = x_ref[...] * m_ref[...]


def kernel(x, psm, mask):
    B, L, C, H, W = x.shape
    S = H * W
    masks = _compute_masks(psm).reshape(B, L - 1, S)
    gate = (mask[:, 1:] != 0).astype(jnp.float32)[:, :, None]
    m_full = jnp.concatenate(
        [jnp.ones((B, 1, S), jnp.float32), masks * gate], axis=1
    ).reshape(B, L, 1, S)
    xr = x.reshape(B, L, C, S)
    cb = 32
    out = pl.pallas_call(
        _apply_kernel,
        grid=(B, L, C // cb),
        in_specs=[
            pl.BlockSpec((1, 1, cb, S), lambda b, l, c: (b, l, c, 0)),
            pl.BlockSpec((1, 1, 1, S), lambda b, l, c: (b, l, 0, 0)),
        ],
        out_specs=pl.BlockSpec((1, 1, cb, S), lambda b, l, c: (b, l, c, 0)),
        out_shape=jax.ShapeDtypeStruct((B, L, C, S), x.dtype),
    )(xr, m_full)
    return out.reshape(B, L, C, H, W)
